# Initial kernel scaffold; baseline (speedup 1.0000x reference)
import jax
import jax.numpy as jnp
from jax.experimental import pallas as pl


def kernel(latents, embedding):
    raise NotImplementedError("write your pallas kernel here")


# speedup vs baseline: 4.8547x; 4.8547x over previous
import functools

import jax
import jax.numpy as jnp
from jax.experimental import pallas as pl
from jax.experimental.pallas import tpu as pltpu

K = 8192
D = 256
N = 8192
BETA = 0.25
BK = 256
KT = K // BK


def _argmin_kernel(flat_ref, emb_ref, idx_ref, sumf2_ref,
                   minval0_ref, minidx0_ref, minval1_ref, minidx1_ref):
    k = pl.program_id(0)

    @pl.when(k == 0)
    def _init():
        f = flat_ref[:]
        sumf2_ref[:] = jnp.sum(f * f, axis=1, keepdims=True)
        minval0_ref[:] = jnp.full((N, 1), jnp.inf, jnp.float32)
        minidx0_ref[:] = jnp.zeros((N, 1), jnp.int32)
        minval1_ref[:] = jnp.full((N, 1), jnp.inf, jnp.float32)
        minidx1_ref[:] = jnp.zeros((N, 1), jnp.int32)

    e = emb_ref[:]
    mm = jax.lax.dot_general(flat_ref[:].astype(jnp.bfloat16),
                             e.astype(jnp.bfloat16), (((1,), (1,)), ((), ())),
                             preferred_element_type=jnp.float32)
    e2 = jnp.sum(e * e, axis=1)[None, :]
    dist = (sumf2_ref[:] - 2.0 * mm) + e2
    tmin = jnp.min(dist, axis=1, keepdims=True)
    ii = jax.lax.broadcasted_iota(jnp.int32, dist.shape, 1)
    tidx = jnp.min(jnp.where(dist == tmin, ii, K), axis=1, keepdims=True) + k * BK

    @pl.when(k < KT // 2)
    def _upd0():
        upd = tmin < minval0_ref[:]
        minidx0_ref[:] = jnp.where(upd, tidx, minidx0_ref[:])
        minval0_ref[:] = jnp.where(upd, tmin, minval0_ref[:])

    @pl.when(k >= KT // 2)
    def _upd1():
        upd = tmin < minval1_ref[:]
        minidx1_ref[:] = jnp.where(upd, tidx, minidx1_ref[:])
        minval1_ref[:] = jnp.where(upd, tmin, minval1_ref[:])

    @pl.when(k == KT - 1)
    def _emit():
        m0_bf16 = minval0_ref[:].astype(jnp.bfloat16).astype(jnp.float32)
        take1 = minval1_ref[:] < m0_bf16
        idx_ref[:] = jnp.where(take1, minidx1_ref[:], minidx0_ref[:])


def _onehot_kernel(idx_ref, emb_ref, oh_ref, counts_ref, zq_ref, acc_ref):
    k = pl.program_id(0)
    ii = jax.lax.broadcasted_iota(jnp.int32, (N, BK), 1) + k * BK
    oh = (ii == idx_ref[:]).astype(jnp.float32)
    oh_ref[:] = oh
    counts_ref[:, pl.ds(k * BK, BK)] = jnp.sum(oh, axis=0, keepdims=True)
    part = jax.lax.dot_general(oh, emb_ref[:], (((1,), (0,)), ((), ())),
                               preferred_element_type=jnp.float32)

    @pl.when(k == 0)
    def _init():
        acc_ref[:] = jnp.zeros((N, D), jnp.float32)

    acc_ref[:] = acc_ref[:] + part

    @pl.when(k == KT - 1)
    def _emit():
        zq_ref[:] = acc_ref[:]


def _finish_kernel(flat_ref, zq_ref, counts_ref, loss_ref, ppx_ref, zqout_ref):
    f = flat_ref[:]
    q = zq_ref[:]
    d = q - f
    m = jnp.mean(d * d)
    loss_ref[:] = (m + BETA * m).reshape(1, 1)
    zqout_ref[:] = f + (q - f)
    p = counts_ref[:] * (1.0 / N)
    ent = jnp.sum(p * jnp.log(p + 1e-10))
    ppx_ref[:] = jnp.exp(-ent).reshape(1, 1)


@functools.partial(jax.jit, static_argnames=())
def kernel(latents, embedding):
    lat = jnp.transpose(latents, (0, 2, 3, 1))
    flat = lat.reshape(-1, D)

    idx = pl.pallas_call(
        _argmin_kernel,
        grid=(KT,),
        in_specs=[
            pl.BlockSpec((N, D), lambda k: (0, 0)),
            pl.BlockSpec((BK, D), lambda k: (k, 0)),
        ],
        out_specs=pl.BlockSpec((N, 1), lambda k: (0, 0)),
        out_shape=jax.ShapeDtypeStruct((N, 1), jnp.int32),
        scratch_shapes=[
            pltpu.VMEM((N, 1), jnp.float32),
            pltpu.VMEM((N, 1), jnp.float32),
            pltpu.VMEM((N, 1), jnp.int32),
            pltpu.VMEM((N, 1), jnp.float32),
            pltpu.VMEM((N, 1), jnp.int32),
        ],
    )(flat, embedding)

    min_embed, counts, zq = pl.pallas_call(
        _onehot_kernel,
        grid=(KT,),
        in_specs=[
            pl.BlockSpec((N, 1), lambda k: (0, 0)),
            pl.BlockSpec((BK, D), lambda k: (k, 0)),
        ],
        out_specs=[
            pl.BlockSpec((N, BK), lambda k: (0, k)),
            pl.BlockSpec((1, K), lambda k: (0, 0)),
            pl.BlockSpec((N, D), lambda k: (0, 0)),
        ],
        out_shape=[
            jax.ShapeDtypeStruct((N, K), jnp.float32),
            jax.ShapeDtypeStruct((1, K), jnp.float32),
            jax.ShapeDtypeStruct((N, D), jnp.float32),
        ],
        scratch_shapes=[pltpu.VMEM((N, D), jnp.float32)],
    )(idx, embedding)

    loss, ppx, zq_out = pl.pallas_call(
        _finish_kernel,
        in_specs=[
            pl.BlockSpec((N, D), lambda: (0, 0)),
            pl.BlockSpec((N, D), lambda: (0, 0)),
            pl.BlockSpec((1, K), lambda: (0, 0)),
        ],
        out_specs=[
            pl.BlockSpec((1, 1), lambda: (0, 0)),
            pl.BlockSpec((1, 1), lambda: (0, 0)),
            pl.BlockSpec((N, D), lambda: (0, 0)),
        ],
        out_shape=[
            jax.ShapeDtypeStruct((1, 1), jnp.float32),
            jax.ShapeDtypeStruct((1, 1), jnp.float32),
            jax.ShapeDtypeStruct((N, D), jnp.float32),
        ],
    )(flat, zq, counts)

    z_q = jnp.transpose(zq_out.reshape(lat.shape), (0, 3, 1, 2))
    return (loss.reshape(()), z_q, ppx.reshape(()), min_embed, idx)


# speedup vs baseline: 5.5049x; 1.1339x over previous
import functools

import jax
import jax.numpy as jnp
from jax.experimental import pallas as pl
from jax.experimental.pallas import tpu as pltpu

K = 8192
D = 256
N = 8192
BETA = 0.25
BK = 256
KT = K // BK


def _argmin_kernel(flat_ref, emb_ref, idx_ref, sumf2_ref,
                   minval0_ref, minidx0_ref, minval1_ref, minidx1_ref,
                   iota_ref):
    k = pl.program_id(0)

    @pl.when(k == 0)
    def _init():
        f = flat_ref[:]
        sumf2_ref[:] = jnp.sum(f * f, axis=1, keepdims=True)
        minval0_ref[:] = jnp.full((N, 1), jnp.inf, jnp.float32)
        minidx0_ref[:] = jnp.zeros((N, 1), jnp.float32)
        minval1_ref[:] = jnp.full((N, 1), jnp.inf, jnp.float32)
        minidx1_ref[:] = jnp.zeros((N, 1), jnp.float32)
        iota_ref[:] = jax.lax.broadcasted_iota(
            jnp.int32, (1, BK), 1).astype(jnp.float32)

    e = emb_ref[:]
    mm2 = jax.lax.dot_general(flat_ref[:].astype(jnp.bfloat16),
                              (e * -2.0).astype(jnp.bfloat16),
                              (((1,), (1,)), ((), ())),
                              preferred_element_type=jnp.float32)
    e2 = jnp.sum(e * e, axis=1)[None, :]
    dist = (sumf2_ref[:] + mm2) + e2
    tmin = jnp.min(dist, axis=1, keepdims=True)
    kbase = (k * BK).astype(jnp.float32)
    tidx = jnp.min(jnp.where(dist == tmin, iota_ref[:], float(K)), axis=1,
                   keepdims=True) + kbase

    @pl.when(k < KT // 2)
    def _upd0():
        upd = tmin < minval0_ref[:]
        minidx0_ref[:] = jnp.where(upd, tidx, minidx0_ref[:])
        minval0_ref[:] = jnp.where(upd, tmin, minval0_ref[:])

    @pl.when(k >= KT // 2)
    def _upd1():
        upd = tmin < minval1_ref[:]
        minidx1_ref[:] = jnp.where(upd, tidx, minidx1_ref[:])
        minval1_ref[:] = jnp.where(upd, tmin, minval1_ref[:])

    @pl.when(k == KT - 1)
    def _emit():
        m0_bf16 = minval0_ref[:].astype(jnp.bfloat16).astype(jnp.float32)
        take1 = minval1_ref[:] < m0_bf16
        idx_ref[:] = jnp.where(take1, minidx1_ref[:],
                               minidx0_ref[:]).astype(jnp.int32)


def _onehot_kernel(idx_ref, emb_ref, oh_ref, counts_ref, zq_ref, acc_ref):
    k = pl.program_id(0)
    ii = jax.lax.broadcasted_iota(jnp.int32, (N, BK), 1) + k * BK
    oh = (ii == idx_ref[:]).astype(jnp.float32)
    oh_ref[:] = oh
    counts_ref[:, pl.ds(k * BK, BK)] = jnp.sum(oh, axis=0, keepdims=True)
    part = jax.lax.dot_general(oh.astype(jnp.bfloat16),
                               emb_ref[:].astype(jnp.bfloat16),
                               (((1,), (0,)), ((), ())),
                               preferred_element_type=jnp.float32)

    @pl.when(k == 0)
    def _init():
        acc_ref[:] = jnp.zeros((N, D), jnp.float32)

    acc_ref[:] = acc_ref[:] + part

    @pl.when(k == KT - 1)
    def _emit():
        zq_ref[:] = acc_ref[:]


def _finish_kernel(flat_ref, zq_ref, counts_ref, loss_ref, ppx_ref, zqout_ref):
    f = flat_ref[:]
    q = zq_ref[:]
    d = q - f
    m = jnp.mean(d * d)
    loss_ref[:] = (m + BETA * m).reshape(1, 1)
    zqout_ref[:] = f + (q - f)
    p = counts_ref[:] * (1.0 / N)
    ent = jnp.sum(p * jnp.log(p + 1e-10))
    ppx_ref[:] = jnp.exp(-ent).reshape(1, 1)


@functools.partial(jax.jit, static_argnames=())
def kernel(latents, embedding):
    lat = jnp.transpose(latents, (0, 2, 3, 1))
    flat = lat.reshape(-1, D)

    idx = pl.pallas_call(
        _argmin_kernel,
        grid=(KT,),
        in_specs=[
            pl.BlockSpec((N, D), lambda k: (0, 0)),
            pl.BlockSpec((BK, D), lambda k: (k, 0)),
        ],
        out_specs=pl.BlockSpec((N, 1), lambda k: (0, 0)),
        out_shape=jax.ShapeDtypeStruct((N, 1), jnp.int32),
        scratch_shapes=[
            pltpu.VMEM((N, 1), jnp.float32),
            pltpu.VMEM((N, 1), jnp.float32),
            pltpu.VMEM((N, 1), jnp.float32),
            pltpu.VMEM((N, 1), jnp.float32),
            pltpu.VMEM((N, 1), jnp.float32),
            pltpu.VMEM((1, BK), jnp.float32),
        ],
    )(flat, embedding)

    min_embed, counts, zq = pl.pallas_call(
        _onehot_kernel,
        grid=(KT,),
        in_specs=[
            pl.BlockSpec((N, 1), lambda k: (0, 0)),
            pl.BlockSpec((BK, D), lambda k: (k, 0)),
        ],
        out_specs=[
            pl.BlockSpec((N, BK), lambda k: (0, k)),
            pl.BlockSpec((1, K), lambda k: (0, 0)),
            pl.BlockSpec((N, D), lambda k: (0, 0)),
        ],
        out_shape=[
            jax.ShapeDtypeStruct((N, K), jnp.float32),
            jax.ShapeDtypeStruct((1, K), jnp.float32),
            jax.ShapeDtypeStruct((N, D), jnp.float32),
        ],
        scratch_shapes=[pltpu.VMEM((N, D), jnp.float32)],
    )(idx, embedding)

    loss, ppx, zq_out = pl.pallas_call(
        _finish_kernel,
        in_specs=[
            pl.BlockSpec((N, D), lambda: (0, 0)),
            pl.BlockSpec((N, D), lambda: (0, 0)),
            pl.BlockSpec((1, K), lambda: (0, 0)),
        ],
        out_specs=[
            pl.BlockSpec((1, 1), lambda: (0, 0)),
            pl.BlockSpec((1, 1), lambda: (0, 0)),
            pl.BlockSpec((N, D), lambda: (0, 0)),
        ],
        out_shape=[
            jax.ShapeDtypeStruct((1, 1), jnp.float32),
            jax.ShapeDtypeStruct((1, 1), jnp.float32),
            jax.ShapeDtypeStruct((N, D), jnp.float32),
        ],
    )(flat, zq, counts)

    z_q = jnp.transpose(zq_out.reshape(lat.shape), (0, 3, 1, 2))
    return (loss.reshape(()), z_q, ppx.reshape(()), min_embed, idx)


# speedup vs baseline: 5.5768x; 1.0131x over previous
import functools

import jax
import jax.numpy as jnp
from jax import lax
from jax.experimental import pallas as pl
from jax.experimental.pallas import tpu as pltpu
from jax.experimental.pallas import tpu_sc as plsc

K = 8192
D = 256
N = 8192
BETA = 0.25
BK = 256
KT = K // BK


def _argmin_kernel(flat_ref, emb_ref, idx_ref, sumf2_ref,
                   minval0_ref, minidx0_ref, minval1_ref, minidx1_ref,
                   iota_ref):
    k = pl.program_id(0)

    @pl.when(k == 0)
    def _init():
        f = flat_ref[:]
        sumf2_ref[:] = jnp.sum(f * f, axis=1, keepdims=True)
        minval0_ref[:] = jnp.full((N, 1), jnp.inf, jnp.float32)
        minidx0_ref[:] = jnp.zeros((N, 1), jnp.float32)
        minval1_ref[:] = jnp.full((N, 1), jnp.inf, jnp.float32)
        minidx1_ref[:] = jnp.zeros((N, 1), jnp.float32)
        iota_ref[:] = jax.lax.broadcasted_iota(
            jnp.int32, (1, BK), 1).astype(jnp.float32)

    e = emb_ref[:]
    mm2 = jax.lax.dot_general(flat_ref[:].astype(jnp.bfloat16),
                              (e * -2.0).astype(jnp.bfloat16),
                              (((1,), (1,)), ((), ())),
                              preferred_element_type=jnp.float32)
    e2 = jnp.sum(e * e, axis=1)[None, :]
    dist = (sumf2_ref[:] + mm2) + e2
    tmin = jnp.min(dist, axis=1, keepdims=True)
    kbase = (k * BK).astype(jnp.float32)
    tidx = jnp.min(jnp.where(dist == tmin, iota_ref[:], float(K)), axis=1,
                   keepdims=True) + kbase

    @pl.when(k < KT // 2)
    def _upd0():
        upd = tmin < minval0_ref[:]
        minidx0_ref[:] = jnp.where(upd, tidx, minidx0_ref[:])
        minval0_ref[:] = jnp.where(upd, tmin, minval0_ref[:])

    @pl.when(k >= KT // 2)
    def _upd1():
        upd = tmin < minval1_ref[:]
        minidx1_ref[:] = jnp.where(upd, tidx, minidx1_ref[:])
        minval1_ref[:] = jnp.where(upd, tmin, minval1_ref[:])

    @pl.when(k == KT - 1)
    def _emit():
        m0_bf16 = minval0_ref[:].astype(jnp.bfloat16).astype(jnp.float32)
        take1 = minval1_ref[:] < m0_bf16
        idx_ref[:] = jnp.where(take1, minidx1_ref[:],
                               minidx0_ref[:]).astype(jnp.int32)


def _onehot_kernel(idx_ref, oh_ref, counts_ref):
    k = pl.program_id(0)
    ii = jax.lax.broadcasted_iota(jnp.int32, (N, BK), 1) + k * BK
    oh = (ii == idx_ref[:]).astype(jnp.float32)
    oh_ref[:] = oh
    counts_ref[:, pl.ds(k * BK, BK)] = jnp.sum(oh, axis=0, keepdims=True)


_SC_INFO = plsc.get_sparse_core_info()
_NW = _SC_INFO.num_cores * _SC_INFO.num_subcores
_BPW = N // _NW
_CH = 64


def _zq_gather_kernel(emb_hbm, idx_hbm, out_hbm, idx_v, rows_v, sem):
    wid = lax.axis_index("s") * _SC_INFO.num_cores + lax.axis_index("c")
    base = wid * _BPW
    for c in range(_BPW // _CH):
        off = base + c * _CH
        pltpu.sync_copy(idx_hbm.at[pl.ds(off, _CH)], idx_v)
        pltpu.async_copy(emb_hbm.at[idx_v], rows_v, sem).wait()
        pltpu.sync_copy(rows_v, out_hbm.at[pl.ds(off, _CH)])


def _finish_kernel(flat_ref, zq_ref, counts_ref, loss_ref, ppx_ref, zqout_ref):
    f = flat_ref[:]
    q = zq_ref[:]
    d = q - f
    m = jnp.mean(d * d)
    loss_ref[:] = (m + BETA * m).reshape(1, 1)
    zqout_ref[:] = f + (q - f)
    p = counts_ref[:] * (1.0 / N)
    ent = jnp.sum(p * jnp.log(p + 1e-10))
    ppx_ref[:] = jnp.exp(-ent).reshape(1, 1)


@functools.partial(jax.jit, static_argnames=())
def kernel(latents, embedding):
    lat = jnp.transpose(latents, (0, 2, 3, 1))
    flat = lat.reshape(-1, D)

    idx = pl.pallas_call(
        _argmin_kernel,
        grid=(KT,),
        in_specs=[
            pl.BlockSpec((N, D), lambda k: (0, 0)),
            pl.BlockSpec((BK, D), lambda k: (k, 0)),
        ],
        out_specs=pl.BlockSpec((N, 1), lambda k: (0, 0)),
        out_shape=jax.ShapeDtypeStruct((N, 1), jnp.int32),
        scratch_shapes=[
            pltpu.VMEM((N, 1), jnp.float32),
            pltpu.VMEM((N, 1), jnp.float32),
            pltpu.VMEM((N, 1), jnp.float32),
            pltpu.VMEM((N, 1), jnp.float32),
            pltpu.VMEM((N, 1), jnp.float32),
            pltpu.VMEM((1, BK), jnp.float32),
        ],
    )(flat, embedding)

    min_embed, counts = pl.pallas_call(
        _onehot_kernel,
        grid=(KT,),
        in_specs=[
            pl.BlockSpec((N, 1), lambda k: (0, 0)),
        ],
        out_specs=[
            pl.BlockSpec((N, BK), lambda k: (0, k)),
            pl.BlockSpec((1, K), lambda k: (0, 0)),
        ],
        out_shape=[
            jax.ShapeDtypeStruct((N, K), jnp.float32),
            jax.ShapeDtypeStruct((1, K), jnp.float32),
        ],
    )(idx)

    zq = pl.kernel(
        _zq_gather_kernel,
        mesh=plsc.VectorSubcoreMesh(core_axis_name="c", subcore_axis_name="s"),
        out_type=jax.ShapeDtypeStruct((N, D), jnp.float32),
        scratch_types=[
            pltpu.VMEM((_CH,), jnp.int32),
            pltpu.VMEM((_CH, D), jnp.float32),
            pltpu.SemaphoreType.DMA,
        ],
    )(embedding, idx.reshape(N))

    loss, ppx, zq_out = pl.pallas_call(
        _finish_kernel,
        in_specs=[
            pl.BlockSpec((N, D), lambda: (0, 0)),
            pl.BlockSpec((N, D), lambda: (0, 0)),
            pl.BlockSpec((1, K), lambda: (0, 0)),
        ],
        out_specs=[
            pl.BlockSpec((1, 1), lambda: (0, 0)),
            pl.BlockSpec((1, 1), lambda: (0, 0)),
            pl.BlockSpec((N, D), lambda: (0, 0)),
        ],
        out_shape=[
            jax.ShapeDtypeStruct((1, 1), jnp.float32),
            jax.ShapeDtypeStruct((1, 1), jnp.float32),
            jax.ShapeDtypeStruct((N, D), jnp.float32),
        ],
    )(flat, zq, counts)

    z_q = jnp.transpose(zq_out.reshape(lat.shape), (0, 3, 1, 2))
    return (loss.reshape(()), z_q, ppx.reshape(()), min_embed, idx)


# speedup vs baseline: 6.8687x; 1.2317x over previous
import functools

import jax
import jax.numpy as jnp
from jax import lax
from jax.experimental import pallas as pl
from jax.experimental.pallas import tpu as pltpu
from jax.experimental.pallas import tpu_sc as plsc

K = 8192
D = 256
N = 8192
BETA = 0.25
BK = 512
KT = K // BK
BKO = 256
KTO = K // BKO


def _argmin_kernel(flat_ref, emb_ref, idx_ref, sumf2_ref,
                   minval0_ref, minidx0_ref, minval1_ref, minidx1_ref,
                   iota_ref):
    k = pl.program_id(0)

    @pl.when(k == 0)
    def _init():
        f = flat_ref[:]
        sumf2_ref[:] = jnp.sum(f * f, axis=1, keepdims=True)
        minval0_ref[:] = jnp.full((N, 1), jnp.inf, jnp.float32)
        minidx0_ref[:] = jnp.zeros((N, 1), jnp.float32)
        minval1_ref[:] = jnp.full((N, 1), jnp.inf, jnp.float32)
        minidx1_ref[:] = jnp.zeros((N, 1), jnp.float32)
        iota_ref[:] = jax.lax.broadcasted_iota(
            jnp.int32, (1, BK), 1).astype(jnp.float32)

    e = emb_ref[:]
    mm2 = jax.lax.dot_general(flat_ref[:].astype(jnp.bfloat16),
                              (e * -2.0).astype(jnp.bfloat16),
                              (((1,), (1,)), ((), ())),
                              preferred_element_type=jnp.float32)
    e2 = jnp.sum(e * e, axis=1)[None, :]
    dist = (sumf2_ref[:] + mm2) + e2
    tmin = jnp.min(dist, axis=1, keepdims=True)
    kbase = (k * BK).astype(jnp.float32)
    tidx = jnp.min(jnp.where(dist == tmin, iota_ref[:], float(K)), axis=1,
                   keepdims=True) + kbase

    @pl.when(k < KT // 2)
    def _upd0():
        upd = tmin < minval0_ref[:]
        minidx0_ref[:] = jnp.where(upd, tidx, minidx0_ref[:])
        minval0_ref[:] = jnp.where(upd, tmin, minval0_ref[:])

    @pl.when(k >= KT // 2)
    def _upd1():
        upd = tmin < minval1_ref[:]
        minidx1_ref[:] = jnp.where(upd, tidx, minidx1_ref[:])
        minval1_ref[:] = jnp.where(upd, tmin, minval1_ref[:])

    @pl.when(k == KT - 1)
    def _emit():
        m0_bf16 = minval0_ref[:].astype(jnp.bfloat16).astype(jnp.float32)
        take1 = minval1_ref[:] < m0_bf16
        idx_ref[:] = jnp.where(take1, minidx1_ref[:],
                               minidx0_ref[:]).astype(jnp.int32)


def _onehot_kernel(idx_ref, oh_ref, counts_ref):
    k = pl.program_id(0)
    ii = jax.lax.broadcasted_iota(jnp.int32, (N, BKO), 1) + k * BKO
    oh = (ii == idx_ref[:]).astype(jnp.float32)
    oh_ref[:] = oh
    counts_ref[:, pl.ds(k * BKO, BKO)] = jnp.sum(oh, axis=0, keepdims=True)


_SC_INFO = plsc.get_sparse_core_info()
_NW = _SC_INFO.num_cores * _SC_INFO.num_subcores
_BPW = N // _NW
_CH = 64


def _zq_gather_kernel(emb_hbm, idx_hbm, out_hbm, idx_v, rows_v, sem):
    wid = lax.axis_index("s") * _SC_INFO.num_cores + lax.axis_index("c")
    base = wid * _BPW
    for c in range(_BPW // _CH):
        off = base + c * _CH
        pltpu.sync_copy(idx_hbm.at[pl.ds(off, _CH)], idx_v)
        pltpu.async_copy(emb_hbm.at[idx_v], rows_v, sem).wait()
        pltpu.sync_copy(rows_v, out_hbm.at[pl.ds(off, _CH)])


def _finish_kernel(flat_ref, zq_ref, counts_ref, loss_ref, ppx_ref, zqout_ref):
    f = flat_ref[:]
    q = zq_ref[:]
    d = q - f
    m = jnp.mean(d * d)
    loss_ref[:] = (m + BETA * m).reshape(1, 1)
    zqout_ref[:] = f + (q - f)
    p = counts_ref[:] * (1.0 / N)
    ent = jnp.sum(p * jnp.log(p + 1e-10))
    ppx_ref[:] = jnp.exp(-ent).reshape(1, 1)


@functools.partial(jax.jit, static_argnames=())
def kernel(latents, embedding):
    lat = jnp.transpose(latents, (0, 2, 3, 1))
    flat = lat.reshape(-1, D)

    idx = pl.pallas_call(
        _argmin_kernel,
        grid=(KT,),
        in_specs=[
            pl.BlockSpec((N, D), lambda k: (0, 0)),
            pl.BlockSpec((BK, D), lambda k: (k, 0)),
        ],
        out_specs=pl.BlockSpec((N, 1), lambda k: (0, 0)),
        out_shape=jax.ShapeDtypeStruct((N, 1), jnp.int32),
        scratch_shapes=[
            pltpu.VMEM((N, 1), jnp.float32),
            pltpu.VMEM((N, 1), jnp.float32),
            pltpu.VMEM((N, 1), jnp.float32),
            pltpu.VMEM((N, 1), jnp.float32),
            pltpu.VMEM((N, 1), jnp.float32),
            pltpu.VMEM((1, BK), jnp.float32),
        ],
    )(flat, embedding)

    min_embed, counts = pl.pallas_call(
        _onehot_kernel,
        grid=(KTO,),
        in_specs=[
            pl.BlockSpec((N, 1), lambda k: (0, 0)),
        ],
        out_specs=[
            pl.BlockSpec((N, BKO), lambda k: (0, k)),
            pl.BlockSpec((1, K), lambda k: (0, 0)),
        ],
        out_shape=[
            jax.ShapeDtypeStruct((N, K), jnp.float32),
            jax.ShapeDtypeStruct((1, K), jnp.float32),
        ],
    )(idx)

    zq = pl.kernel(
        _zq_gather_kernel,
        mesh=plsc.VectorSubcoreMesh(core_axis_name="c", subcore_axis_name="s"),
        out_type=jax.ShapeDtypeStruct((N, D), jnp.float32),
        scratch_types=[
            pltpu.VMEM((_CH,), jnp.int32),
            pltpu.VMEM((_CH, D), jnp.float32),
            pltpu.SemaphoreType.DMA,
        ],
    )(embedding, idx.reshape(N))

    loss, ppx, zq_out = pl.pallas_call(
        _finish_kernel,
        in_specs=[
            pl.BlockSpec((N, D), lambda: (0, 0)),
            pl.BlockSpec((N, D), lambda: (0, 0)),
            pl.BlockSpec((1, K), lambda: (0, 0)),
        ],
        out_specs=[
            pl.BlockSpec((1, 1), lambda: (0, 0)),
            pl.BlockSpec((1, 1), lambda: (0, 0)),
            pl.BlockSpec((N, D), lambda: (0, 0)),
        ],
        out_shape=[
            jax.ShapeDtypeStruct((1, 1), jnp.float32),
            jax.ShapeDtypeStruct((1, 1), jnp.float32),
            jax.ShapeDtypeStruct((N, D), jnp.float32),
        ],
    )(flat, zq, counts)

    z_q = jnp.transpose(zq_out.reshape(lat.shape), (0, 3, 1, 2))
    return (loss.reshape(()), z_q, ppx.reshape(()), min_embed, idx)


# speedup vs baseline: 7.2204x; 1.0512x over previous
import functools

import jax
import jax.numpy as jnp
from jax import lax
from jax.experimental import pallas as pl
from jax.experimental.pallas import tpu as pltpu
from jax.experimental.pallas import tpu_sc as plsc

K = 8192
D = 256
N = 8192
BETA = 0.25
BK = 1024
KT = K // BK
BKO = 256
KTO = K // BKO


def _argmin_kernel(flat_ref, emb_ref, idx_ref, acc_ref, iota_ref):
    k = pl.program_id(0)

    @pl.when(k == 0)
    def _init():
        f = flat_ref[:]
        acc_ref[:, 0:1] = jnp.sum(f * f, axis=1, keepdims=True)
        acc_ref[:, 1:2] = jnp.full((N, 1), jnp.inf, jnp.float32)
        acc_ref[:, 2:3] = jnp.zeros((N, 1), jnp.float32)
        acc_ref[:, 3:4] = jnp.full((N, 1), jnp.inf, jnp.float32)
        acc_ref[:, 4:5] = jnp.zeros((N, 1), jnp.float32)
        iota_ref[:] = jax.lax.broadcasted_iota(
            jnp.int32, (1, BK), 1).astype(jnp.float32)

    e = emb_ref[:]
    mm2 = jax.lax.dot_general(flat_ref[:].astype(jnp.bfloat16),
                              (e * -2.0).astype(jnp.bfloat16),
                              (((1,), (1,)), ((), ())),
                              preferred_element_type=jnp.float32)
    e2 = jnp.sum(e * e, axis=1)[None, :]
    dist = (acc_ref[:, 0:1] + mm2) + e2
    tmin = jnp.min(dist, axis=1, keepdims=True)
    kbase = (k * BK).astype(jnp.float32)
    tidx = jnp.min(jnp.where(dist == tmin, iota_ref[:], float(K)), axis=1,
                   keepdims=True) + kbase

    @pl.when(k < KT // 2)
    def _upd0():
        upd = tmin < acc_ref[:, 1:2]
        acc_ref[:, 2:3] = jnp.where(upd, tidx, acc_ref[:, 2:3])
        acc_ref[:, 1:2] = jnp.where(upd, tmin, acc_ref[:, 1:2])

    @pl.when(k >= KT // 2)
    def _upd1():
        upd = tmin < acc_ref[:, 3:4]
        acc_ref[:, 4:5] = jnp.where(upd, tidx, acc_ref[:, 4:5])
        acc_ref[:, 3:4] = jnp.where(upd, tmin, acc_ref[:, 3:4])

    @pl.when(k == KT - 1)
    def _emit():
        m0_bf16 = acc_ref[:, 1:2].astype(jnp.bfloat16).astype(jnp.float32)
        take1 = acc_ref[:, 3:4] < m0_bf16
        idx_ref[:] = jnp.where(take1, acc_ref[:, 4:5],
                               acc_ref[:, 2:3]).astype(jnp.int32)


def _onehot_kernel(idx_ref, oh_ref, counts_ref):
    k = pl.program_id(0)
    ii = jax.lax.broadcasted_iota(jnp.int32, (N, BKO), 1) + k * BKO
    oh = (ii == idx_ref[:]).astype(jnp.float32)
    oh_ref[:] = oh
    counts_ref[:, pl.ds(k * BKO, BKO)] = jnp.sum(oh, axis=0, keepdims=True)


_SC_INFO = plsc.get_sparse_core_info()
_NW = _SC_INFO.num_cores * _SC_INFO.num_subcores
_BPW = N // _NW
_CH = 64


def _zq_gather_kernel(emb_hbm, idx_hbm, out_hbm, idx_v, rows_v, sem):
    wid = lax.axis_index("s") * _SC_INFO.num_cores + lax.axis_index("c")
    base = wid * _BPW
    for c in range(_BPW // _CH):
        off = base + c * _CH
        pltpu.sync_copy(idx_hbm.at[pl.ds(off, _CH)], idx_v)
        pltpu.async_copy(emb_hbm.at[idx_v], rows_v, sem).wait()
        pltpu.sync_copy(rows_v, out_hbm.at[pl.ds(off, _CH)])


def _finish_kernel(flat_ref, zq_ref, counts_ref, loss_ref, ppx_ref, zqout_ref):
    f = flat_ref[:]
    q = zq_ref[:]
    d = q - f
    m = jnp.mean(d * d)
    loss_ref[:] = (m + BETA * m).reshape(1, 1)
    zqout_ref[:] = f + (q - f)
    p = counts_ref[:] * (1.0 / N)
    ent = jnp.sum(p * jnp.log(p + 1e-10))
    ppx_ref[:] = jnp.exp(-ent).reshape(1, 1)


@functools.partial(jax.jit, static_argnames=())
def kernel(latents, embedding):
    lat = jnp.transpose(latents, (0, 2, 3, 1))
    flat = lat.reshape(-1, D)

    idx = pl.pallas_call(
        _argmin_kernel,
        grid=(KT,),
        in_specs=[
            pl.BlockSpec((N, D), lambda k: (0, 0)),
            pl.BlockSpec((BK, D), lambda k: (k, 0)),
        ],
        out_specs=pl.BlockSpec((N, 1), lambda k: (0, 0)),
        out_shape=jax.ShapeDtypeStruct((N, 1), jnp.int32),
        scratch_shapes=[
            pltpu.VMEM((N, 8), jnp.float32),
            pltpu.VMEM((1, BK), jnp.float32),
        ],
    )(flat, embedding)

    min_embed, counts = pl.pallas_call(
        _onehot_kernel,
        grid=(KTO,),
        in_specs=[
            pl.BlockSpec((N, 1), lambda k: (0, 0)),
        ],
        out_specs=[
            pl.BlockSpec((N, BKO), lambda k: (0, k)),
            pl.BlockSpec((1, K), lambda k: (0, 0)),
        ],
        out_shape=[
            jax.ShapeDtypeStruct((N, K), jnp.float32),
            jax.ShapeDtypeStruct((1, K), jnp.float32),
        ],
    )(idx)

    zq = pl.kernel(
        _zq_gather_kernel,
        mesh=plsc.VectorSubcoreMesh(core_axis_name="c", subcore_axis_name="s"),
        out_type=jax.ShapeDtypeStruct((N, D), jnp.float32),
        scratch_types=[
            pltpu.VMEM((_CH,), jnp.int32),
            pltpu.VMEM((_CH, D), jnp.float32),
            pltpu.SemaphoreType.DMA,
        ],
    )(embedding, idx.reshape(N))

    loss, ppx, zq_out = pl.pallas_call(
        _finish_kernel,
        in_specs=[
            pl.BlockSpec((N, D), lambda: (0, 0)),
            pl.BlockSpec((N, D), lambda: (0, 0)),
            pl.BlockSpec((1, K), lambda: (0, 0)),
        ],
        out_specs=[
            pl.BlockSpec((1, 1), lambda: (0, 0)),
            pl.BlockSpec((1, 1), lambda: (0, 0)),
            pl.BlockSpec((N, D), lambda: (0, 0)),
        ],
        out_shape=[
            jax.ShapeDtypeStruct((1, 1), jnp.float32),
            jax.ShapeDtypeStruct((1, 1), jnp.float32),
            jax.ShapeDtypeStruct((N, D), jnp.float32),
        ],
    )(flat, zq, counts)

    z_q = jnp.transpose(zq_out.reshape(lat.shape), (0, 3, 1, 2))
    return (loss.reshape(()), z_q, ppx.reshape(()), min_embed, idx)
